# baseline (speedup 1.0000x reference)
import jax
import jax.numpy as jnp
from jax import lax
from jax.experimental import pallas as pl
from jax.experimental.pallas import tpu as pltpu
from jax.experimental.pallas import tpu_sc as plsc

N_CLUSTERS = 512
DIM = 64
N_POS = 50
N_SAMPLES = 16384

_INFO = plsc.get_sparse_core_info()
NC = _INFO.num_cores
NS = _INFO.num_subcores
NW = NC * NS

S_PER_W = N_SAMPLES // NW
STEP = 128
N_Q = S_PER_W // STEP
TROW = STEP + 1
LANES = 16


def _body(xt_hbm, table_hbm, out_hbm, idx_v, gbuf, tbuf, table_sp,
          gsem0, gsem1, osem0, osem1):
    sid = lax.axis_index("s")
    wid = sid * NC + lax.axis_index("c")
    ibase = wid * S_PER_W
    blk0 = wid * N_Q

    @pl.when(sid == 0)
    def _():
        pltpu.sync_copy(table_hbm, table_sp)

    pltpu.sync_copy(xt_hbm.at[:, pl.ds(blk0, N_Q)], idx_v)
    plsc.subcore_barrier()

    gsems = (gsem0, gsem1)
    osems = (osem0, osem1)

    def fire_gather(t, r):
        pltpu.async_copy(
            table_sp.at[idx_v.at[t // N_Q, t % N_Q]], gbuf.at[r], gsems[r]
        )

    def drain_gather(r):
        pltpu.make_async_copy(
            table_hbm.at[pl.ds(0, STEP)], gbuf.at[r], gsems[r]
        ).wait()

    def transpose(r):
        iotas = [lax.iota(jnp.int32, LANES) + b * LANES
                 for b in range(DIM // LANES)]

        def rows(i4, carry):
            for di in range(4):
                i = i4 * 4 + di
                ivec = jnp.full((LANES,), i, jnp.int32)
                for b in range(DIM // LANES):
                    vals = gbuf[r, i, pl.ds(b * LANES, LANES)]
                    plsc.store_scatter(tbuf.at[r], [iotas[b], ivec], vals)
            return carry

        lax.fori_loop(0, STEP // 4, rows, 0)

    def fire_out(t, r):
        pltpu.async_copy(
            tbuf.at[r, :, pl.ds(0, STEP)],
            out_hbm.at[t // N_Q, :, pl.ds(ibase + (t % N_Q) * STEP, STEP)],
            osems[r],
        )

    def drain_out(r):
        pltpu.make_async_copy(
            out_hbm.at[0, :, pl.ds(0, STEP)], tbuf.at[r, :, pl.ds(0, STEP)],
            osems[r],
        ).wait()

    n_steps = N_POS * N_Q

    fire_gather(0, 0)
    fire_gather(1, 1)

    drain_gather(0)
    transpose(0)
    fire_out(0, 0)
    fire_gather(2, 0)

    drain_gather(1)
    transpose(1)
    fire_out(1, 1)
    fire_gather(3, 1)

    def step_pair(tt, carry):
        for r in range(2):
            t = 2 * tt + 2 + r
            drain_gather(r)
            drain_out(r)
            transpose(r)
            fire_out(t, r)

            @pl.when(t + 2 < n_steps)
            def _():
                fire_gather(t + 2, r)

        return carry

    lax.fori_loop(0, (n_steps - 2) // 2, step_pair, 0)

    drain_out(0)
    drain_out(1)


@jax.jit
def kernel(x, embedding_weight):
    xt = x.astype(jnp.int32).T.reshape(N_POS, N_SAMPLES // STEP, STEP)

    mesh = plsc.VectorSubcoreMesh(core_axis_name="c", subcore_axis_name="s")
    out = pl.kernel(
        _body,
        out_type=jax.ShapeDtypeStruct((N_POS, DIM, N_SAMPLES), jnp.float32),
        mesh=mesh,
        scratch_types=[
            pltpu.VMEM((N_POS, N_Q, STEP), jnp.int32),
            pltpu.VMEM((2, STEP, DIM), jnp.float32),
            pltpu.VMEM((2, DIM, TROW), jnp.float32),
            pltpu.VMEM_SHARED((N_CLUSTERS, DIM), jnp.float32),
            pltpu.SemaphoreType.DMA,
            pltpu.SemaphoreType.DMA,
            pltpu.SemaphoreType.DMA,
            pltpu.SemaphoreType.DMA,
        ],
        compiler_params=pltpu.CompilerParams(
            use_tc_tiling_on_sc=False, needs_layout_passes=False
        ),
    )(xt, embedding_weight)
    return jnp.transpose(out, (2, 0, 1))


# speedup vs baseline: 1.5773x; 1.5773x over previous
import jax
import jax.numpy as jnp
from jax import lax
from jax.experimental import pallas as pl
from jax.experimental.pallas import tpu as pltpu
from jax.experimental.pallas import tpu_sc as plsc

N_CLUSTERS = 512
DIM = 64
N_POS = 50
N_SAMPLES = 16384

_INFO = plsc.get_sparse_core_info()
NC = _INFO.num_cores
NS = _INFO.num_subcores
NW = NC * NS

S_PER_W = N_SAMPLES // NW
STEP = 128
N_Q = S_PER_W // STEP
TROW = STEP + 1
LANES = 16


def _body(xt_hbm, table_hbm, out_hbm, idx_v, gbuf, tbuf, table_sp,
          gsem0, gsem1, osem0, osem1):
    sid = lax.axis_index("s")
    wid = sid * NC + lax.axis_index("c")
    ibase = wid * S_PER_W
    blk0 = wid * N_Q

    @pl.when(sid == 0)
    def _():
        pltpu.sync_copy(table_hbm, table_sp)

    pltpu.sync_copy(xt_hbm.at[:, pl.ds(blk0, N_Q)], idx_v)
    plsc.subcore_barrier()

    gsems = (gsem0, gsem1)
    osems = (osem0, osem1)

    def fire_gather(t, r):
        pltpu.async_copy(
            table_sp.at[idx_v.at[t // N_Q, t % N_Q]], gbuf.at[r], gsems[r]
        )

    def drain_gather(r):
        pltpu.make_async_copy(
            table_hbm.at[pl.ds(0, STEP)], gbuf.at[r], gsems[r]
        ).wait()

    def transpose(r):
        iota = lax.iota(jnp.int32, LANES)
        zero = jnp.zeros((LANES,), jnp.int32)
        idx_hi = [(iota + b * LANES) // 8 for b in range(DIM // LANES)]
        idx_lo = [(iota + b * LANES) % 8 for b in range(DIM // LANES)]

        def rows(i4, carry):
            for di in range(4):
                i = i4 * 4 + di
                ivec = jnp.full((LANES,), i, jnp.int32)
                for b in range(DIM // LANES):
                    vals = gbuf[r, i, pl.ds(b * LANES, LANES)]
                    plsc.store_scatter(
                        tbuf.at[r], [idx_hi[b], zero, idx_lo[b], ivec], vals
                    )
            return carry

        lax.fori_loop(0, STEP // 4, rows, 0)

    def fire_out(t, r):
        pltpu.async_copy(
            tbuf.at[r, :, :, :, pl.ds(0, STEP)],
            out_hbm.at[t // N_Q, :,
                       pl.ds(ibase // STEP + (t % N_Q), 1)],
            osems[r],
        )

    def drain_out(r):
        pltpu.make_async_copy(
            out_hbm.at[0, :, pl.ds(0, 1)], tbuf.at[r, :, :, :, pl.ds(0, STEP)],
            osems[r],
        ).wait()

    n_steps = N_POS * N_Q

    fire_gather(0, 0)
    fire_gather(1, 1)

    drain_gather(0)
    transpose(0)
    fire_out(0, 0)
    fire_gather(2, 0)

    drain_gather(1)
    transpose(1)
    fire_out(1, 1)
    fire_gather(3, 1)

    def step_pair(tt, carry):
        for r in range(2):
            t = 2 * tt + 2 + r
            drain_gather(r)
            drain_out(r)
            transpose(r)
            fire_out(t, r)

            @pl.when(t + 2 < n_steps)
            def _():
                fire_gather(t + 2, r)

        return carry

    lax.fori_loop(0, (n_steps - 2) // 2, step_pair, 0)

    drain_out(0)
    drain_out(1)


@jax.jit
def kernel(x, embedding_weight):
    xt = x.astype(jnp.int32).T.reshape(N_POS, N_SAMPLES // STEP, STEP)

    mesh = plsc.VectorSubcoreMesh(core_axis_name="c", subcore_axis_name="s")
    out = pl.kernel(
        _body,
        out_type=jax.ShapeDtypeStruct(
            (N_POS, DIM // 8, N_SAMPLES // STEP, 8, STEP), jnp.float32
        ),
        mesh=mesh,
        scratch_types=[
            pltpu.VMEM((N_POS, N_Q, STEP), jnp.int32),
            pltpu.VMEM((2, STEP, DIM), jnp.float32),
            pltpu.VMEM((2, DIM // 8, 1, 8, TROW), jnp.float32),
            pltpu.VMEM_SHARED((N_CLUSTERS, DIM), jnp.float32),
            pltpu.SemaphoreType.DMA,
            pltpu.SemaphoreType.DMA,
            pltpu.SemaphoreType.DMA,
            pltpu.SemaphoreType.DMA,
        ],
        compiler_params=pltpu.CompilerParams(
            use_tc_tiling_on_sc=False, needs_layout_passes=False
        ),
    )(xt, embedding_weight)
    return jnp.transpose(out, (2, 4, 0, 1, 3)).reshape(
        N_SAMPLES, N_POS, DIM
    )


# speedup vs baseline: 1.5901x; 1.0081x over previous
import jax
import jax.numpy as jnp
from jax import lax
from jax.experimental import pallas as pl
from jax.experimental.pallas import tpu as pltpu
from jax.experimental.pallas import tpu_sc as plsc

N_CLUSTERS = 512
DIM = 64
N_POS = 50
N_SAMPLES = 16384

_INFO = plsc.get_sparse_core_info()
NC = _INFO.num_cores
NS = _INFO.num_subcores
NW = NC * NS

S_PER_W = N_SAMPLES // NW
STEP = 128
N_Q = S_PER_W // STEP
TROW = STEP + 1
LANES = 16


def _body(xt_hbm, table_hbm, out_hbm, idx_v, gbuf, tbuf, table_sp,
          gsem0, gsem1, osem0, osem1):
    sid = lax.axis_index("s")
    wid = sid * NC + lax.axis_index("c")
    ibase = wid * S_PER_W
    blk0 = wid * N_Q

    @pl.when(sid == 0)
    def _():
        pltpu.sync_copy(table_hbm, table_sp)

    pltpu.sync_copy(xt_hbm.at[:, pl.ds(blk0, N_Q)], idx_v)
    plsc.subcore_barrier()

    gsems = (gsem0, gsem1)
    osems = (osem0, osem1)

    def fire_gather(t, r):
        pltpu.async_copy(
            table_sp.at[idx_v.at[t // N_Q, t % N_Q]], gbuf.at[r], gsems[r]
        )

    def drain_gather(r):
        pltpu.make_async_copy(
            table_hbm.at[pl.ds(0, STEP)], gbuf.at[r], gsems[r]
        ).wait()

    def transpose(r):
        iota = lax.iota(jnp.int32, LANES)
        zero = jnp.zeros((LANES,), jnp.int32)
        one = jnp.ones((LANES,), jnp.int32)
        idx_hi = [(iota + b * LANES) // 8 for b in range(DIM // LANES)]
        idx_lo = [(iota + b * LANES) % 8 for b in range(DIM // LANES)]

        def rows(i4, ivec0):
            ivec = ivec0
            for di in range(4):
                i = i4 * 4 + di
                for b in range(DIM // LANES):
                    vals = gbuf[r, i, pl.ds(b * LANES, LANES)]
                    plsc.store_scatter(
                        tbuf.at[r], [idx_hi[b], zero, idx_lo[b], ivec], vals
                    )
                ivec = ivec + one
            return ivec

        lax.fori_loop(0, STEP // 4, rows, zero)

    def fire_out(t, r):
        pltpu.async_copy(
            tbuf.at[r, :, :, :, pl.ds(0, STEP)],
            out_hbm.at[t // N_Q, :,
                       pl.ds(ibase // STEP + (t % N_Q), 1)],
            osems[r],
        )

    def drain_out(r):
        pltpu.make_async_copy(
            out_hbm.at[0, :, pl.ds(0, 1)], tbuf.at[r, :, :, :, pl.ds(0, STEP)],
            osems[r],
        ).wait()

    n_steps = N_POS * N_Q

    fire_gather(0, 0)
    fire_gather(1, 1)

    drain_gather(0)
    transpose(0)
    fire_out(0, 0)
    fire_gather(2, 0)

    drain_gather(1)
    transpose(1)
    fire_out(1, 1)
    fire_gather(3, 1)

    def step_pair(tt, carry):
        for r in range(2):
            t = 2 * tt + 2 + r
            drain_gather(r)
            drain_out(r)
            transpose(r)
            fire_out(t, r)

            @pl.when(t + 2 < n_steps)
            def _():
                fire_gather(t + 2, r)

        return carry

    lax.fori_loop(0, (n_steps - 2) // 2, step_pair, 0)

    drain_out(0)
    drain_out(1)


@jax.jit
def kernel(x, embedding_weight):
    xt = x.astype(jnp.int32).T.reshape(N_POS, N_SAMPLES // STEP, STEP)

    mesh = plsc.VectorSubcoreMesh(core_axis_name="c", subcore_axis_name="s")
    out = pl.kernel(
        _body,
        out_type=jax.ShapeDtypeStruct(
            (N_POS, DIM // 8, N_SAMPLES // STEP, 8, STEP), jnp.float32
        ),
        mesh=mesh,
        scratch_types=[
            pltpu.VMEM((N_POS, N_Q, STEP), jnp.int32),
            pltpu.VMEM((2, STEP, DIM), jnp.float32),
            pltpu.VMEM((2, DIM // 8, 1, 8, TROW), jnp.float32),
            pltpu.VMEM_SHARED((N_CLUSTERS, DIM), jnp.float32),
            pltpu.SemaphoreType.DMA,
            pltpu.SemaphoreType.DMA,
            pltpu.SemaphoreType.DMA,
            pltpu.SemaphoreType.DMA,
        ],
        compiler_params=pltpu.CompilerParams(
            use_tc_tiling_on_sc=False, needs_layout_passes=False
        ),
    )(xt, embedding_weight)
    return jnp.transpose(out, (2, 4, 0, 1, 3)).reshape(
        N_SAMPLES, N_POS, DIM
    )


# speedup vs baseline: 1.5910x; 1.0006x over previous
import jax
import jax.numpy as jnp
from jax import lax
from jax.experimental import pallas as pl
from jax.experimental.pallas import tpu as pltpu
from jax.experimental.pallas import tpu_sc as plsc

N_CLUSTERS = 512
DIM = 64
N_POS = 50
N_SAMPLES = 16384

_INFO = plsc.get_sparse_core_info()
NC = _INFO.num_cores
NS = _INFO.num_subcores
NW = NC * NS

S_PER_W = N_SAMPLES // NW
STEP = 128
N_Q = S_PER_W // STEP
TROW = STEP + 8
LANES = 16


def _body(xt_hbm, table_hbm, out_hbm, idx_v, gbuf, tbuf, table_sp,
          gsem0, gsem1, osem0, osem1):
    sid = lax.axis_index("s")
    wid = sid * NC + lax.axis_index("c")
    ibase = wid * S_PER_W
    blk0 = wid * N_Q

    @pl.when(sid == 0)
    def _():
        pltpu.sync_copy(table_hbm, table_sp)

    pltpu.sync_copy(xt_hbm.at[:, pl.ds(blk0, N_Q)], idx_v)
    plsc.subcore_barrier()

    gsems = (gsem0, gsem1)
    osems = (osem0, osem1)

    def fire_gather(t, r):
        pltpu.async_copy(
            table_sp.at[idx_v.at[t // N_Q, t % N_Q]], gbuf.at[r], gsems[r]
        )

    def drain_gather(r):
        pltpu.make_async_copy(
            table_hbm.at[pl.ds(0, STEP)], gbuf.at[r], gsems[r]
        ).wait()

    def transpose(r):
        iota = lax.iota(jnp.int32, LANES)
        zero = jnp.zeros((LANES,), jnp.int32)
        one = jnp.ones((LANES,), jnp.int32)
        idx_hi = [(iota + b * LANES) // 8 for b in range(DIM // LANES)]
        idx_lo = [(iota + b * LANES) % 8 for b in range(DIM // LANES)]

        def rows(i4, ivec0):
            ivec = ivec0
            for di in range(4):
                i = i4 * 4 + di
                for b in range(DIM // LANES):
                    vals = gbuf[r, i, pl.ds(b * LANES, LANES)]
                    plsc.store_scatter(
                        tbuf.at[r], [idx_hi[b], zero, idx_lo[b], ivec], vals
                    )
                ivec = ivec + one
            return ivec

        lax.fori_loop(0, STEP // 4, rows, zero)

    def fire_out(t, r):
        pltpu.async_copy(
            tbuf.at[r, :, :, :, pl.ds(0, STEP)],
            out_hbm.at[t // N_Q, :,
                       pl.ds(ibase // STEP + (t % N_Q), 1)],
            osems[r],
        )

    def drain_out(r):
        pltpu.make_async_copy(
            out_hbm.at[0, :, pl.ds(0, 1)], tbuf.at[r, :, :, :, pl.ds(0, STEP)],
            osems[r],
        ).wait()

    n_steps = N_POS * N_Q

    fire_gather(0, 0)
    fire_gather(1, 1)

    drain_gather(0)
    transpose(0)
    fire_out(0, 0)
    fire_gather(2, 0)

    drain_gather(1)
    transpose(1)
    fire_out(1, 1)
    fire_gather(3, 1)

    def step_pair(tt, carry):
        for r in range(2):
            t = 2 * tt + 2 + r
            drain_gather(r)
            drain_out(r)
            transpose(r)
            fire_out(t, r)

            @pl.when(t + 2 < n_steps)
            def _():
                fire_gather(t + 2, r)

        return carry

    lax.fori_loop(0, (n_steps - 2) // 2, step_pair, 0)

    drain_out(0)
    drain_out(1)


@jax.jit
def kernel(x, embedding_weight):
    xt = x.astype(jnp.int32).T.reshape(N_POS, N_SAMPLES // STEP, STEP)

    mesh = plsc.VectorSubcoreMesh(core_axis_name="c", subcore_axis_name="s")
    out = pl.kernel(
        _body,
        out_type=jax.ShapeDtypeStruct(
            (N_POS, DIM // 8, N_SAMPLES // STEP, 8, STEP), jnp.float32
        ),
        mesh=mesh,
        scratch_types=[
            pltpu.VMEM((N_POS, N_Q, STEP), jnp.int32),
            pltpu.VMEM((2, STEP, DIM), jnp.float32),
            pltpu.VMEM((2, DIM // 8, 1, 8, TROW), jnp.float32),
            pltpu.VMEM_SHARED((N_CLUSTERS, DIM), jnp.float32),
            pltpu.SemaphoreType.DMA,
            pltpu.SemaphoreType.DMA,
            pltpu.SemaphoreType.DMA,
            pltpu.SemaphoreType.DMA,
        ],
        compiler_params=pltpu.CompilerParams(
            use_tc_tiling_on_sc=False, needs_layout_passes=False
        ),
    )(xt, embedding_weight)
    return jnp.transpose(out, (2, 4, 0, 1, 3)).reshape(
        N_SAMPLES, N_POS, DIM
    )


# speedup vs baseline: 4.2988x; 2.7019x over previous
import jax
import jax.numpy as jnp
from jax import lax
from jax.experimental import pallas as pl
from jax.experimental.pallas import tpu as pltpu
from jax.experimental.pallas import tpu_sc as plsc

N_CLUSTERS = 512
DIM = 64
N_POS = 50
N_SAMPLES = 16384

_INFO = plsc.get_sparse_core_info()
NC = _INFO.num_cores
NS = _INFO.num_subcores
NW = NC * NS

S_PER_W = N_SAMPLES // NW
STEP = 128
N_Q = S_PER_W // STEP
TROW = STEP + 8
LANES = 16


def _body(xt_hbm, table_hbm, out_hbm, idx_v, gbuf, tbuf, table_sp,
          gsem0, gsem1, osem0, osem1):
    sid = lax.axis_index("s")
    wid = sid * NC + lax.axis_index("c")
    ibase = wid * S_PER_W
    blk0 = wid * N_Q

    @pl.when(sid == 0)
    def _():
        pltpu.sync_copy(table_hbm, table_sp)

    pltpu.sync_copy(xt_hbm.at[:, pl.ds(blk0, N_Q)], idx_v)
    plsc.subcore_barrier()

    gsems = (gsem0, gsem1)
    osems = (osem0, osem1)

    def fire_gather(t, r):
        pltpu.async_copy(
            table_sp.at[idx_v.at[t // N_Q, t % N_Q]], gbuf.at[r], gsems[r]
        )

    def drain_gather(r):
        pltpu.make_async_copy(
            table_hbm.at[pl.ds(0, STEP)], gbuf.at[r], gsems[r]
        ).wait()

    def transpose(r):
        iota = lax.iota(jnp.int32, LANES)
        zero = jnp.zeros((LANES,), jnp.int32)
        one = jnp.ones((LANES,), jnp.int32)
        idx_hi = [(iota + b * LANES) // 8 for b in range(DIM // LANES)]
        idx_lo = [(iota + b * LANES) % 8 for b in range(DIM // LANES)]

        del one

        @plsc.parallel_loop(0, STEP // 4, 1, unroll=2)
        def rows(i4):
            for di in range(4):
                i = i4 * 4 + di
                ivec = jnp.full((LANES,), i, jnp.int32)
                for b in range(DIM // LANES):
                    vals = gbuf[r, i, pl.ds(b * LANES, LANES)]
                    plsc.store_scatter(
                        tbuf.at[r], [idx_hi[b], zero, idx_lo[b], ivec], vals
                    )

    def fire_out(t, r):
        pltpu.async_copy(
            tbuf.at[r, :, :, :, pl.ds(0, STEP)],
            out_hbm.at[t // N_Q, :,
                       pl.ds(ibase // STEP + (t % N_Q), 1)],
            osems[r],
        )

    def drain_out(r):
        pltpu.make_async_copy(
            out_hbm.at[0, :, pl.ds(0, 1)], tbuf.at[r, :, :, :, pl.ds(0, STEP)],
            osems[r],
        ).wait()

    n_steps = N_POS * N_Q

    fire_gather(0, 0)
    fire_gather(1, 1)

    drain_gather(0)
    transpose(0)
    fire_out(0, 0)
    fire_gather(2, 0)

    drain_gather(1)
    transpose(1)
    fire_out(1, 1)
    fire_gather(3, 1)

    def step_pair(tt, carry):
        for r in range(2):
            t = 2 * tt + 2 + r
            drain_gather(r)
            drain_out(r)
            transpose(r)
            fire_out(t, r)

            @pl.when(t + 2 < n_steps)
            def _():
                fire_gather(t + 2, r)

        return carry

    lax.fori_loop(0, (n_steps - 2) // 2, step_pair, 0)

    drain_out(0)
    drain_out(1)


@jax.jit
def kernel(x, embedding_weight):
    xt = x.astype(jnp.int32).T.reshape(N_POS, N_SAMPLES // STEP, STEP)

    mesh = plsc.VectorSubcoreMesh(core_axis_name="c", subcore_axis_name="s")
    out = pl.kernel(
        _body,
        out_type=jax.ShapeDtypeStruct(
            (N_POS, DIM // 8, N_SAMPLES // STEP, 8, STEP), jnp.float32
        ),
        mesh=mesh,
        scratch_types=[
            pltpu.VMEM((N_POS, N_Q, STEP), jnp.int32),
            pltpu.VMEM((2, STEP, DIM), jnp.float32),
            pltpu.VMEM((2, DIM // 8, 1, 8, TROW), jnp.float32),
            pltpu.VMEM_SHARED((N_CLUSTERS, DIM), jnp.float32),
            pltpu.SemaphoreType.DMA,
            pltpu.SemaphoreType.DMA,
            pltpu.SemaphoreType.DMA,
            pltpu.SemaphoreType.DMA,
        ],
        compiler_params=pltpu.CompilerParams(
            use_tc_tiling_on_sc=False, needs_layout_passes=False
        ),
    )(xt, embedding_weight)
    return jnp.transpose(out, (2, 4, 0, 1, 3)).reshape(
        N_SAMPLES, N_POS, DIM
    )
